# Initial kernel scaffold; baseline (speedup 1.0000x reference)
import jax
import jax.numpy as jnp
from jax.experimental import pallas as pl


def kernel(h, W1, b1, W2, b2, W3, b3, edge_index):
    raise NotImplementedError("write your pallas kernel here")


# speedup vs baseline: 7.5115x; 7.5115x over previous
import functools

import jax
import jax.numpy as jnp
from jax import lax
from jax.experimental import pallas as pl
from jax.experimental.pallas import tpu as pltpu
from jax.experimental.pallas import tpu_sc as plsc

N = 10000
NP = 10240
E = 320000
D = 128

NC = 2
NS = 16
NW = NC * NS
K = 80
ROWS = E // K
RPT = ROWS // NW
NPT = NP // NS
DEGW = 16


def _zero_2d(ref, nrows, ncols):
    def row(r, carry):
        for j in range(ncols // 16):
            ref[r, pl.ds(j * 16, 16)] = jnp.zeros((16,), jnp.float32)
        return carry
    lax.fori_loop(0, nrows, row, 0)


def _sc_agg_body(compute_deg, h_hbm, src_hbm, dst_hbm, *refs):
    if compute_deg:
        part_hbm, degp_hbm, srcbuf, dstbuf, rows, onesb, acc, degacc, sem = refs
    else:
        part_hbm, srcbuf, dstbuf, rows, acc, sem = refs

    c = lax.axis_index("c")
    s = lax.axis_index("s")
    wid = c * NS + s

    pltpu.sync_copy(src_hbm.at[wid], srcbuf)
    pltpu.sync_copy(dst_hbm.at[wid], dstbuf)

    _zero_2d(rows, K, D)
    for r in range(NPT // K):
        pltpu.sync_copy(rows, acc.at[pl.ds(s * NPT + r * K, K)])
    if compute_deg:
        _zero_2d(onesb, K, DEGW)
        for r in range(NPT // K):
            pltpu.sync_copy(onesb, degacc.at[pl.ds(s * NPT + r * K, K)])

        def orow(r, carry):
            onesb[r, pl.ds(0, 16)] = jnp.ones((16,), jnp.float32)
            return carry
        lax.fori_loop(0, K, orow, 0)
    plsc.subcore_barrier()

    def step(j, carry):
        pltpu.async_copy(h_hbm.at[srcbuf.at[j]], rows, sem).wait()
        pltpu.sync_copy(rows, acc.at[dstbuf.at[j]], add=True)
        if compute_deg:
            pltpu.sync_copy(onesb, degacc.at[dstbuf.at[j]], add=True)
        return carry
    lax.fori_loop(0, RPT, step, 0)

    plsc.subcore_barrier()

    pltpu.sync_copy(acc.at[pl.ds(s * NPT, NPT)],
                    part_hbm.at[c, pl.ds(s * NPT, NPT)])
    if compute_deg:
        pltpu.sync_copy(degacc.at[pl.ds(s * NPT, NPT)],
                        degp_hbm.at[c, pl.ds(s * NPT, NPT)])


def _make_sc_agg(compute_deg):
    mesh = plsc.VectorSubcoreMesh(core_axis_name="c", subcore_axis_name="s")
    out_type = [jax.ShapeDtypeStruct((NC, NP, D), jnp.float32)]
    scratch = [
        pltpu.VMEM((RPT, K), jnp.int32),
        pltpu.VMEM((RPT, K), jnp.int32),
        pltpu.VMEM((K, D), jnp.float32),
    ]
    if compute_deg:
        out_type.append(jax.ShapeDtypeStruct((NC, NP, DEGW), jnp.float32))
        scratch.append(pltpu.VMEM((K, DEGW), jnp.float32))
    scratch.append(pltpu.VMEM_SHARED((NP, D), jnp.float32))
    if compute_deg:
        scratch.append(pltpu.VMEM_SHARED((NP, DEGW), jnp.float32))
    scratch.append(pltpu.SemaphoreType.DMA)

    return pl.kernel(
        functools.partial(_sc_agg_body, compute_deg),
        out_type=tuple(out_type),
        mesh=mesh,
        scratch_types=tuple(scratch),
        compiler_params=pltpu.CompilerParams(use_tc_tiling_on_sc=False),
    )


def _tc_block_body(residual, p_ref, deg_ref, h_ref, w_ref, b_ref, o_ref):
    agg = (p_ref[0] + p_ref[1]) / jnp.maximum(deg_ref[...], 1.0)
    y = jnp.dot(agg, w_ref[...], preferred_element_type=jnp.float32)
    y = jnp.maximum(y + b_ref[...], 0.0)
    if residual:
        y = y + h_ref[...]
    o_ref[...] = y


def _make_tc_block(residual):
    BN = 1024
    return pl.pallas_call(
        functools.partial(_tc_block_body, residual),
        grid=(NP // BN,),
        in_specs=[
            pl.BlockSpec((NC, BN, D), lambda i: (0, i, 0)),
            pl.BlockSpec((BN, 1), lambda i: (i, 0)),
            pl.BlockSpec((BN, D), lambda i: (i, 0)),
            pl.BlockSpec((D, D), lambda i: (0, 0)),
            pl.BlockSpec((1, D), lambda i: (0, 0)),
        ],
        out_specs=pl.BlockSpec((BN, D), lambda i: (i, 0)),
        out_shape=jax.ShapeDtypeStruct((NP, D), jnp.float32),
    )


def kernel(h, W1, b1, W2, b2, W3, b3, edge_index):
    src = edge_index[0].astype(jnp.int32).reshape(NW, RPT, K)
    dst = edge_index[1].astype(jnp.int32).reshape(NW, RPT, K)

    agg_deg = _make_sc_agg(True)
    agg = _make_sc_agg(False)
    blk_res = _make_tc_block(True)
    blk_last = _make_tc_block(False)

    hp = jnp.pad(h, ((0, NP - N), (0, 0)))
    part, degp = agg_deg(hp, src, dst)
    deg = (degp[0, :, 0] + degp[1, :, 0]).reshape(NP, 1)

    h1 = blk_res(part, deg, hp, W1, b1.reshape(1, D))
    (part,) = agg(h1, src, dst)
    h2 = blk_res(part, deg, h1, W2, b2.reshape(1, D))
    (part,) = agg(h2, src, dst)
    h3 = blk_last(part, deg, h2, W3, b3.reshape(1, D))
    return h3[:N]


# speedup vs baseline: 9.9662x; 1.3268x over previous
import functools

import jax
import jax.numpy as jnp
from jax import lax
from jax.experimental import pallas as pl
from jax.experimental.pallas import tpu as pltpu
from jax.experimental.pallas import tpu_sc as plsc

N = 10000
NP = 10240
E = 320000
D = 128

NC = 2
NS = 16
NW = NC * NS
K = 80
ROWS = E // K
RPT = ROWS // NW
NPT = NP // NS
DEGW = 16


def _zero_2d(ref, nrows, ncols):
    def row(r, carry):
        for j in range(ncols // 16):
            ref[r, pl.ds(j * 16, 16)] = jnp.zeros((16,), jnp.float32)
        return carry
    lax.fori_loop(0, nrows, row, 0)


def _sc_agg_body(compute_deg, h_hbm, src_hbm, dst_hbm, *refs):
    if compute_deg:
        (part_hbm, degp_hbm, srcbuf, dstbuf, rows0, onesb, acc, degacc,
         sem0) = refs
        bufs, sems = (rows0,), (sem0,)
    else:
        part_hbm, srcbuf, dstbuf, rows0, rows1, acc, sem0, sem1 = refs
        bufs, sems = (rows0, rows1), (sem0, sem1)
    nbuf = len(bufs)
    rows = rows0

    c = lax.axis_index("c")
    s = lax.axis_index("s")
    wid = c * NS + s

    pltpu.sync_copy(src_hbm.at[wid], srcbuf)
    pltpu.sync_copy(dst_hbm.at[wid], dstbuf)

    _zero_2d(rows, K, D)
    for r in range(NPT // K):
        pltpu.sync_copy(rows, acc.at[pl.ds(s * NPT + r * K, K)])
    if compute_deg:
        _zero_2d(onesb, K, DEGW)
        for r in range(NPT // K):
            pltpu.sync_copy(onesb, degacc.at[pl.ds(s * NPT + r * K, K)])

        def orow(r, carry):
            onesb[r, pl.ds(0, 16)] = jnp.ones((16,), jnp.float32)
            return carry
        lax.fori_loop(0, K, orow, 0)
    plsc.subcore_barrier()

    def consume(j, b):
        pltpu.make_async_copy(h_hbm.at[srcbuf.at[j]], bufs[b], sems[b]).wait()
        pltpu.sync_copy(bufs[b], acc.at[dstbuf.at[j]], add=True)
        if compute_deg:
            pltpu.sync_copy(onesb, degacc.at[dstbuf.at[j]], add=True)

    for b in range(nbuf):
        pltpu.async_copy(h_hbm.at[srcbuf.at[b]], bufs[b], sems[b])

    def step(i, carry):
        for b in range(nbuf):
            j = i * nbuf + b
            consume(j, b)

            @pl.when(j + nbuf < RPT)
            def _():
                pltpu.async_copy(h_hbm.at[srcbuf.at[j + nbuf]], bufs[b],
                                 sems[b])
        return carry
    lax.fori_loop(0, RPT // nbuf, step, 0)
    for j in range(RPT - RPT % nbuf, RPT):
        consume(j, j % nbuf)

    plsc.subcore_barrier()

    pltpu.sync_copy(acc.at[pl.ds(s * NPT, NPT)],
                    part_hbm.at[c, pl.ds(s * NPT, NPT)])
    if compute_deg:
        pltpu.sync_copy(degacc.at[pl.ds(s * NPT, NPT)],
                        degp_hbm.at[c, pl.ds(s * NPT, NPT)])


def _make_sc_agg(compute_deg):
    mesh = plsc.VectorSubcoreMesh(core_axis_name="c", subcore_axis_name="s")
    out_type = [jax.ShapeDtypeStruct((NC, NP, D), jnp.float32)]
    scratch = [
        pltpu.VMEM((RPT, K), jnp.int32),
        pltpu.VMEM((RPT, K), jnp.int32),
        pltpu.VMEM((K, D), jnp.float32),
    ]
    if compute_deg:
        out_type.append(jax.ShapeDtypeStruct((NC, NP, DEGW), jnp.float32))
        scratch.append(pltpu.VMEM((K, DEGW), jnp.float32))
        scratch.append(pltpu.VMEM_SHARED((NP, D), jnp.float32))
        scratch.append(pltpu.VMEM_SHARED((NP, DEGW), jnp.float32))
        scratch.append(pltpu.SemaphoreType.DMA)
    else:
        scratch.append(pltpu.VMEM((K, D), jnp.float32))
        scratch.append(pltpu.VMEM_SHARED((NP, D), jnp.float32))
        scratch += [pltpu.SemaphoreType.DMA, pltpu.SemaphoreType.DMA]

    return pl.kernel(
        functools.partial(_sc_agg_body, compute_deg),
        out_type=tuple(out_type),
        mesh=mesh,
        scratch_types=tuple(scratch),
        compiler_params=pltpu.CompilerParams(use_tc_tiling_on_sc=False),
    )


def _tc_block_body(residual, p_ref, deg_ref, h_ref, w_ref, b_ref, o_ref):
    agg = (p_ref[0] + p_ref[1]) / jnp.maximum(deg_ref[...], 1.0)
    y = jnp.dot(agg, w_ref[...], preferred_element_type=jnp.float32)
    y = jnp.maximum(y + b_ref[...], 0.0)
    if residual:
        y = y + h_ref[...]
    o_ref[...] = y


def _make_tc_block(residual):
    BN = 1024
    return pl.pallas_call(
        functools.partial(_tc_block_body, residual),
        grid=(NP // BN,),
        in_specs=[
            pl.BlockSpec((NC, BN, D), lambda i: (0, i, 0)),
            pl.BlockSpec((BN, 1), lambda i: (i, 0)),
            pl.BlockSpec((BN, D), lambda i: (i, 0)),
            pl.BlockSpec((D, D), lambda i: (0, 0)),
            pl.BlockSpec((1, D), lambda i: (0, 0)),
        ],
        out_specs=pl.BlockSpec((BN, D), lambda i: (i, 0)),
        out_shape=jax.ShapeDtypeStruct((NP, D), jnp.float32),
    )


def kernel(h, W1, b1, W2, b2, W3, b3, edge_index):
    src = edge_index[0].astype(jnp.int32).reshape(NW, RPT, K)
    dst = edge_index[1].astype(jnp.int32).reshape(NW, RPT, K)

    agg_deg = _make_sc_agg(True)
    agg = _make_sc_agg(False)
    blk_res = _make_tc_block(True)
    blk_last = _make_tc_block(False)

    hp = jnp.pad(h, ((0, NP - N), (0, 0)))
    part, degp = agg_deg(hp, src, dst)
    deg = (degp[0, :, 0] + degp[1, :, 0]).reshape(NP, 1)

    h1 = blk_res(part, deg, hp, W1, b1.reshape(1, D))
    (part,) = agg(h1, src, dst)
    h2 = blk_res(part, deg, h1, W2, b2.reshape(1, D))
    (part,) = agg(h2, src, dst)
    h3 = blk_last(part, deg, h2, W3, b3.reshape(1, D))
    return h3[:N]


# speedup vs baseline: 12.2903x; 1.2332x over previous
import functools

import jax
import jax.numpy as jnp
from jax import lax
from jax.experimental import pallas as pl
from jax.experimental.pallas import tpu as pltpu
from jax.experimental.pallas import tpu_sc as plsc

N = 10000
NP = 10240
E = 320000
D = 128

NC = 2
NS = 16
NW = NC * NS
K = 80
ROWS = E // K
RPT = ROWS // NW
NPT = NP // NS
DEGW = 16

_SC_PARAMS = pltpu.CompilerParams(use_tc_tiling_on_sc=False)
_MESH = dict(core_axis_name="c", subcore_axis_name="s")


def _zero_2d(ref, nrows, ncols):
    def row(r, carry):
        for j in range(ncols // 16):
            ref[r, pl.ds(j * 16, 16)] = jnp.zeros((16,), jnp.float32)
        return carry
    lax.fori_loop(0, nrows, row, 0)


def _sc_agg_body(h_hbm, src_hbm, dst_hbm, part_hbm, srcbuf, dstbuf, rows0,
                 rows1, acc, sem0, sem1):
    bufs, sems = (rows0, rows1), (sem0, sem1)
    nbuf = 2

    c = lax.axis_index("c")
    s = lax.axis_index("s")
    wid = c * NS + s

    pltpu.sync_copy(src_hbm.at[wid], srcbuf)
    pltpu.sync_copy(dst_hbm.at[wid], dstbuf)

    _zero_2d(rows0, K, D)
    for r in range(NPT // K):
        pltpu.sync_copy(rows0, acc.at[pl.ds(s * NPT + r * K, K)])
    plsc.subcore_barrier()

    def consume(j, b):
        pltpu.make_async_copy(h_hbm.at[srcbuf.at[j]], bufs[b], sems[b]).wait()
        pltpu.sync_copy(bufs[b], acc.at[dstbuf.at[j]], add=True)

    for b in range(nbuf):
        pltpu.async_copy(h_hbm.at[srcbuf.at[b]], bufs[b], sems[b])

    def step(i, carry):
        for b in range(nbuf):
            j = i * nbuf + b
            consume(j, b)

            @pl.when(j + nbuf < RPT)
            def _():
                pltpu.async_copy(h_hbm.at[srcbuf.at[j + nbuf]], bufs[b],
                                 sems[b])
        return carry
    lax.fori_loop(0, RPT // nbuf, step, 0)
    for j in range(RPT - RPT % nbuf, RPT):
        consume(j, j % nbuf)

    plsc.subcore_barrier()

    pltpu.sync_copy(acc.at[pl.ds(s * NPT, NPT)],
                    part_hbm.at[c, pl.ds(s * NPT, NPT)])


def _make_sc_agg():
    return pl.kernel(
        _sc_agg_body,
        out_type=jax.ShapeDtypeStruct((NC, NP, D), jnp.float32),
        mesh=plsc.VectorSubcoreMesh(**_MESH),
        scratch_types=(
            pltpu.VMEM((RPT, K), jnp.int32),
            pltpu.VMEM((RPT, K), jnp.int32),
            pltpu.VMEM((K, D), jnp.float32),
            pltpu.VMEM((K, D), jnp.float32),
            pltpu.VMEM_SHARED((NP, D), jnp.float32),
            pltpu.SemaphoreType.DMA,
            pltpu.SemaphoreType.DMA,
        ),
        compiler_params=_SC_PARAMS,
    )


def _sc_deg_body(dst_hbm, degp_hbm, dstbuf, onesb, degacc, sem):
    c = lax.axis_index("c")
    s = lax.axis_index("s")
    wid = c * NS + s

    pltpu.sync_copy(dst_hbm.at[wid], dstbuf)

    _zero_2d(onesb, K, DEGW)
    for r in range(NPT // K):
        pltpu.sync_copy(onesb, degacc.at[pl.ds(s * NPT + r * K, K)])

    def orow(r, carry):
        onesb[r, pl.ds(0, 16)] = jnp.ones((16,), jnp.float32)
        return carry
    lax.fori_loop(0, K, orow, 0)
    plsc.subcore_barrier()

    def fire(j, carry):
        pltpu.async_copy(onesb, degacc.at[dstbuf.at[j]], sem, add=True)
        return carry
    lax.fori_loop(0, RPT, fire, 0)

    def drain(j, carry):
        pltpu.make_async_copy(onesb, degacc.at[dstbuf.at[0]], sem).wait()
        return carry
    lax.fori_loop(0, RPT, drain, 0)

    plsc.subcore_barrier()
    pltpu.sync_copy(degacc.at[pl.ds(s * NPT, NPT)],
                    degp_hbm.at[c, pl.ds(s * NPT, NPT)])


def _make_sc_deg():
    return pl.kernel(
        _sc_deg_body,
        out_type=jax.ShapeDtypeStruct((NC, NP, DEGW), jnp.float32),
        mesh=plsc.VectorSubcoreMesh(**_MESH),
        scratch_types=(
            pltpu.VMEM((RPT, K), jnp.int32),
            pltpu.VMEM((K, DEGW), jnp.float32),
            pltpu.VMEM_SHARED((NP, DEGW), jnp.float32),
            pltpu.SemaphoreType.DMA,
        ),
        compiler_params=_SC_PARAMS,
    )


def _tc_block_body(residual, p_ref, deg_ref, h_ref, w_ref, b_ref, o_ref):
    agg = (p_ref[0] + p_ref[1]) / jnp.maximum(deg_ref[...], 1.0)
    y = jnp.dot(agg, w_ref[...], preferred_element_type=jnp.float32)
    y = jnp.maximum(y + b_ref[...], 0.0)
    if residual:
        y = y + h_ref[...]
    o_ref[...] = y


def _make_tc_block(residual):
    BN = 1024
    return pl.pallas_call(
        functools.partial(_tc_block_body, residual),
        grid=(NP // BN,),
        in_specs=[
            pl.BlockSpec((NC, BN, D), lambda i: (0, i, 0)),
            pl.BlockSpec((BN, 1), lambda i: (i, 0)),
            pl.BlockSpec((BN, D), lambda i: (i, 0)),
            pl.BlockSpec((D, D), lambda i: (0, 0)),
            pl.BlockSpec((1, D), lambda i: (0, 0)),
        ],
        out_specs=pl.BlockSpec((BN, D), lambda i: (i, 0)),
        out_shape=jax.ShapeDtypeStruct((NP, D), jnp.float32),
    )


def kernel(h, W1, b1, W2, b2, W3, b3, edge_index):
    src = edge_index[0].astype(jnp.int32).reshape(NW, RPT, K)
    dst = edge_index[1].astype(jnp.int32).reshape(NW, RPT, K)

    deg_k = _make_sc_deg()
    agg = _make_sc_agg()
    blk_res = _make_tc_block(True)
    blk_last = _make_tc_block(False)

    hp = jnp.pad(h, ((0, NP - N), (0, 0)))
    degp = deg_k(dst)
    deg = (degp[0, :, 0] + degp[1, :, 0]).reshape(NP, 1)

    part = agg(hp, src, dst)
    h1 = blk_res(part, deg, hp, W1, b1.reshape(1, D))
    part = agg(h1, src, dst)
    h2 = blk_res(part, deg, h1, W2, b2.reshape(1, D))
    part = agg(h2, src, dst)
    h3 = blk_last(part, deg, h2, W3, b3.reshape(1, D))
    return h3[:N]
